# baseline (speedup 1.0000x reference)
import functools
import jax
import jax.numpy as jnp
from jax import lax
from jax.experimental import pallas as pl
from jax.experimental.pallas import tpu as pltpu
from jax.experimental.pallas import tpu_sc as plsc

D_MODEL = 64
SCALE = 8.0

NC = 2
NS = 16
NW = NC * NS
LANES = 16

SEQ = 200
NSEQ = 4096
IB = 128
RSTRIDE = 64
NBUF = 4


def _transpose_scale(rows_v, obuf):
    iota = lax.iota(jnp.int32, LANES)
    dlv = lax.rem(iota, 8)
    dhvs = [lax.div(iota, 8) + 2 * k for k in range(D_MODEL // LANES)]

    @plsc.parallel_loop(0, IB, unroll=4)
    def _(t):
        tv = jnp.full((LANES,), t, jnp.int32)
        for k in range(D_MODEL // LANES):
            v = rows_v[t, pl.ds(k * LANES, LANES)] * SCALE
            plsc.store_scatter(obuf, [dhvs[k], dlv, tv], v)


def _emb_body(x_hbm, table_hbm, out_hbm, idx_v, *bufs_and_sems):
    rows = bufs_and_sems[:NBUF]
    obufs = bufs_and_sems[NBUF:2 * NBUF]
    gsem = bufs_and_sems[2 * NBUF:3 * NBUF]
    ssem = bufs_and_sems[3 * NBUF:4 * NBUF]

    wid = lax.axis_index("s") * NC + lax.axis_index("c")
    pltpu.sync_copy(x_hbm.at[:, wid], idx_v)

    def fire_gather(j, b):
        pltpu.async_copy(
            table_hbm.at[idx_v.at[j // 8, j % 8]],
            rows[b], gsem[b])

    def wait_gather(j, b):
        pltpu.make_async_copy(
            table_hbm.at[idx_v.at[j // 8, j % 8]],
            rows[b], gsem[b]).wait()

    def fire_scatter(j, b):
        pltpu.async_copy(obufs[b], out_hbm.at[j, :, wid], ssem[b])

    def drain_scatter(b):
        pltpu.make_async_copy(obufs[b], out_hbm.at[0, :, wid], ssem[b]).wait()

    def process(j, b):
        wait_gather(j, b)
        _transpose_scale(rows[b], obufs[b])
        fire_scatter(j, b)

    fire_gather(0, 0)
    fire_gather(1, 1)
    for j in (0, 1):
        fire_gather(j + 2, (j + 2) % NBUF)
        process(j, j % NBUF)

    def outer(jo, _):
        j0 = 2 + jo * NBUF
        for t in range(NBUF):
            j = j0 + t
            b = (2 + t) % NBUF
            bn = (b + 2) % NBUF
            drain_scatter(bn)
            fire_gather(j + 2, bn)
            process(j, b)
        return ()

    lax.fori_loop(0, (SEQ - 4) // NBUF, outer, ())

    for j in (SEQ - 2, SEQ - 1):
        process(j, j % NBUF)
    for b in range(NBUF):
        drain_scatter(b)


def kernel(x, table):
    assert x.shape == (NSEQ, SEQ)
    xv = x.reshape(NW, IB, SEQ // 8, 8).transpose(2, 0, 3, 1)

    mesh = plsc.VectorSubcoreMesh(
        core_axis_name="c", subcore_axis_name="s", num_cores=NC, num_subcores=NS
    )
    run = pl.kernel(
        _emb_body,
        out_type=jax.ShapeDtypeStruct(
            (SEQ, D_MODEL // 8, NW, 8, IB), jnp.float32),
        mesh=mesh,
        scratch_types=(
            [pltpu.VMEM((SEQ // 8, 8, IB), jnp.int32)]
            + [pltpu.VMEM((IB, RSTRIDE), jnp.float32) for _ in range(NBUF)]
            + [pltpu.VMEM((D_MODEL // 8, 8, IB), jnp.float32)
               for _ in range(NBUF)]
            + [pltpu.SemaphoreType.DMA for _ in range(2 * NBUF)]
        ),
        compiler_params=pltpu.CompilerParams(
            use_tc_tiling_on_sc=False, needs_layout_passes=False),
    )
    out6 = run(xv, table)
    return out6.transpose(2, 4, 0, 1, 3).reshape(NSEQ, SEQ, D_MODEL)


# speedup vs baseline: 1.2889x; 1.2889x over previous
import functools
import jax
import jax.numpy as jnp
from jax import lax
from jax.experimental import pallas as pl
from jax.experimental.pallas import tpu as pltpu
from jax.experimental.pallas import tpu_sc as plsc

D_MODEL = 64
DPAD = 128
SCALE = 8.0

NC = 2
NS = 16
NW = NC * NS
LANES = 16

SEQ = 200
SPLIT = 128
NBUF = 3


def _scale_buf(buf):
    @plsc.parallel_loop(0, SEQ, unroll=4)
    def _(r):
        for k in range(DPAD // LANES):
            sl = pl.ds(k * LANES, LANES)
            buf[r, sl] = buf[r, sl] * SCALE


def _emb_body(rows_per_w, x_hbm, table_hbm, out_hbm, idx_v, *bufs_and_sems):
    rows = bufs_and_sems[:NBUF]
    gsem = bufs_and_sems[NBUF:2 * NBUF]
    ssem = bufs_and_sems[2 * NBUF:3 * NBUF]

    wid = lax.axis_index("s") * NC + lax.axis_index("c")
    row0 = wid * rows_per_w
    pltpu.sync_copy(x_hbm.at[pl.ds(row0, rows_per_w)], idx_v)

    def gathers(i, b):
        pltpu.async_copy(
            table_hbm.at[idx_v.at[i, pl.ds(0, SPLIT)]],
            rows[b].at[pl.ds(0, SPLIT)], gsem[b])
        pltpu.async_copy(
            table_hbm.at[idx_v.at[i, pl.ds(SPLIT, SEQ - SPLIT)]],
            rows[b].at[pl.ds(SPLIT, SEQ - SPLIT)], gsem[b])

    def wait_gathers(i, b):
        pltpu.make_async_copy(
            table_hbm.at[idx_v.at[i, pl.ds(0, SPLIT)]],
            rows[b].at[pl.ds(0, SPLIT)], gsem[b]).wait()
        pltpu.make_async_copy(
            table_hbm.at[idx_v.at[i, pl.ds(SPLIT, SEQ - SPLIT)]],
            rows[b].at[pl.ds(SPLIT, SEQ - SPLIT)], gsem[b]).wait()

    def fire_scatter(i, b):
        pltpu.async_copy(rows[b], out_hbm.at[row0 + i], ssem[b])

    def drain_scatter(b):
        pltpu.make_async_copy(rows[b], out_hbm.at[row0], ssem[b]).wait()

    def process(i, b):
        wait_gathers(i, b)
        _scale_buf(rows[b])
        fire_scatter(i, b)

    gathers(0, 0)
    gathers(1, 1)
    for i in (0, 1, 2):
        process(i, i % NBUF)
        gathers(i + 2, (i + 2) % NBUF)

    def outer(io, _):
        i0 = 3 + io * NBUF
        for t in range(NBUF):
            i = i0 + t
            b = t
            drain_scatter(b)
            wait_gathers(i, b)
            _scale_buf(rows[b])
            fire_scatter(i, b)
            gathers(i + 2, (b + 2) % NBUF)
        return ()

    lax.fori_loop(0, (rows_per_w - 5) // NBUF, outer, ())

    for i in (rows_per_w - 2, rows_per_w - 1):
        b = i % NBUF
        drain_scatter(b)
        process(i, b)
    for i in (rows_per_w - 3, rows_per_w - 2, rows_per_w - 1):
        drain_scatter(i % NBUF)


def kernel(x, table):
    n_seq, seq = x.shape
    assert seq == SEQ
    assert n_seq % NW == 0
    rows_per_w = n_seq // NW
    assert (rows_per_w - 5) % NBUF == 0

    t128 = jnp.pad(table, ((0, 0), (0, DPAD - D_MODEL)))

    mesh = plsc.VectorSubcoreMesh(
        core_axis_name="c", subcore_axis_name="s", num_cores=NC, num_subcores=NS
    )
    run = pl.kernel(
        functools.partial(_emb_body, rows_per_w),
        out_type=jax.ShapeDtypeStruct((n_seq, seq, DPAD), jnp.float32),
        mesh=mesh,
        scratch_types=(
            [pltpu.VMEM((rows_per_w, SEQ), jnp.int32)]
            + [pltpu.VMEM((SEQ, DPAD), jnp.float32) for _ in range(NBUF)]
            + [pltpu.SemaphoreType.DMA for _ in range(2 * NBUF)]
        ),
        compiler_params=pltpu.CompilerParams(use_tc_tiling_on_sc=False),
    )
    out128 = run(x, t128)
    return out128[:, :, :D_MODEL]


# speedup vs baseline: 1.2895x; 1.0004x over previous
import functools
import jax
import jax.numpy as jnp
from jax import lax
from jax.experimental import pallas as pl
from jax.experimental.pallas import tpu as pltpu
from jax.experimental.pallas import tpu_sc as plsc

D_MODEL = 64
DPAD = 128
SCALE = 8.0

NC = 2
NS = 16
NW = NC * NS
LANES = 16

SEQ = 200
SPLIT = 128
NBUF = 3


def _scale_buf(buf):
    @plsc.parallel_loop(0, SEQ, unroll=4)
    def _(r):
        for k in range(DPAD // LANES):
            sl = pl.ds(k * LANES, LANES)
            buf[r, sl] = buf[r, sl] * SCALE


def _emb_body(rows_per_w, x_hbm, table_hbm, out_hbm, idx_v, *bufs_and_sems):
    rows = bufs_and_sems[:NBUF]
    gsem = bufs_and_sems[NBUF:2 * NBUF]
    ssem = bufs_and_sems[2 * NBUF:3 * NBUF]

    wid = lax.axis_index("s") * NC + lax.axis_index("c")
    row0 = wid * rows_per_w
    pltpu.sync_copy(x_hbm.at[pl.ds(row0, rows_per_w)], idx_v)

    def gathers(i, b):
        pltpu.async_copy(
            table_hbm.at[idx_v.at[i, pl.ds(0, SPLIT)]],
            rows[b].at[pl.ds(0, SPLIT)], gsem[b])
        pltpu.async_copy(
            table_hbm.at[idx_v.at[i, pl.ds(SPLIT, SEQ - SPLIT)]],
            rows[b].at[pl.ds(SPLIT, SEQ - SPLIT)], gsem[b])

    def wait_gathers(i, b):
        pltpu.make_async_copy(
            table_hbm.at[idx_v.at[i, pl.ds(0, SPLIT)]],
            rows[b].at[pl.ds(0, SPLIT)], gsem[b]).wait()
        pltpu.make_async_copy(
            table_hbm.at[idx_v.at[i, pl.ds(SPLIT, SEQ - SPLIT)]],
            rows[b].at[pl.ds(SPLIT, SEQ - SPLIT)], gsem[b]).wait()

    def fire_scatter(i, b):
        pltpu.async_copy(rows[b], out_hbm.at[row0 + i], ssem[b])

    def drain_scatter(b):
        pltpu.make_async_copy(rows[b], out_hbm.at[row0], ssem[b]).wait()

    def process(i, b):
        wait_gathers(i, b)
        _scale_buf(rows[b])
        fire_scatter(i, b)

    gathers(0, 0)
    gathers(1, 1)
    process(0, 0)
    gathers(2, 2)
    for i in (1, 2):
        process(i, i % NBUF)
        drain_scatter((i + 2) % NBUF)
        gathers(i + 2, (i + 2) % NBUF)

    def outer(io, _):
        i0 = 3 + io * NBUF
        for t in range(NBUF):
            i = i0 + t
            b = t
            wait_gathers(i, b)
            _scale_buf(rows[b])
            fire_scatter(i, b)
            bn = (b + 2) % NBUF
            drain_scatter(bn)
            gathers(i + 2, bn)
        return ()

    lax.fori_loop(0, (rows_per_w - 5) // NBUF, outer, ())

    for i in (rows_per_w - 2, rows_per_w - 1):
        process(i, i % NBUF)
    for i in (rows_per_w - 3, rows_per_w - 2, rows_per_w - 1):
        drain_scatter(i % NBUF)


def kernel(x, table):
    n_seq, seq = x.shape
    assert seq == SEQ
    assert n_seq % NW == 0
    rows_per_w = n_seq // NW
    assert (rows_per_w - 5) % NBUF == 0

    t128 = jnp.pad(table, ((0, 0), (0, DPAD - D_MODEL)))

    mesh = plsc.VectorSubcoreMesh(
        core_axis_name="c", subcore_axis_name="s", num_cores=NC, num_subcores=NS
    )
    run = pl.kernel(
        functools.partial(_emb_body, rows_per_w),
        out_type=jax.ShapeDtypeStruct((n_seq, seq, DPAD), jnp.float32),
        mesh=mesh,
        scratch_types=(
            [pltpu.VMEM((rows_per_w, SEQ), jnp.int32)]
            + [pltpu.VMEM((SEQ, DPAD), jnp.float32) for _ in range(NBUF)]
            + [pltpu.SemaphoreType.DMA for _ in range(2 * NBUF)]
        ),
        compiler_params=pltpu.CompilerParams(use_tc_tiling_on_sc=False),
    )
    out128 = run(x, t128)
    return out128[:, :, :D_MODEL]


# speedup vs baseline: 1.3667x; 1.0599x over previous
import functools
import jax
import jax.numpy as jnp
from jax import lax
from jax.experimental import pallas as pl
from jax.experimental.pallas import tpu as pltpu
from jax.experimental.pallas import tpu_sc as plsc

D_MODEL = 64
DPAD = 128
SCALE = 8.0

NC = 2
NS = 16
NW = NC * NS
LANES = 16

SEQ = 200
SPLIT = 128
NBUF = 3


def _scale_buf(buf):
    @plsc.parallel_loop(0, SEQ, unroll=4)
    def _(r):
        for k in range(D_MODEL // LANES):
            sl = pl.ds(k * LANES, LANES)
            buf[r, sl] = buf[r, sl] * SCALE


def _emb_body(rows_per_w, x_hbm, table_hbm, out_hbm, idx_v, *bufs_and_sems):
    rows = bufs_and_sems[:NBUF]
    gsem = bufs_and_sems[NBUF:2 * NBUF]
    ssem = bufs_and_sems[2 * NBUF:3 * NBUF]

    wid = lax.axis_index("s") * NC + lax.axis_index("c")
    row0 = wid * rows_per_w
    pltpu.sync_copy(x_hbm.at[pl.ds(row0, rows_per_w)], idx_v)

    def gathers(i, b):
        pltpu.async_copy(
            table_hbm.at[idx_v.at[i, pl.ds(0, SPLIT)]],
            rows[b].at[pl.ds(0, SPLIT)], gsem[b])
        pltpu.async_copy(
            table_hbm.at[idx_v.at[i, pl.ds(SPLIT, SEQ - SPLIT)]],
            rows[b].at[pl.ds(SPLIT, SEQ - SPLIT)], gsem[b])

    def wait_gathers(i, b):
        pltpu.make_async_copy(
            table_hbm.at[idx_v.at[i, pl.ds(0, SPLIT)]],
            rows[b].at[pl.ds(0, SPLIT)], gsem[b]).wait()
        pltpu.make_async_copy(
            table_hbm.at[idx_v.at[i, pl.ds(SPLIT, SEQ - SPLIT)]],
            rows[b].at[pl.ds(SPLIT, SEQ - SPLIT)], gsem[b]).wait()

    def fire_scatter(i, b):
        pltpu.async_copy(
            rows[b].at[:, pl.ds(0, D_MODEL)],
            out_hbm.at[row0 + i, :, pl.ds(0, D_MODEL)], ssem[b])

    def drain_scatter(b):
        pltpu.make_async_copy(
            rows[b].at[:, pl.ds(0, D_MODEL)],
            out_hbm.at[row0, :, pl.ds(0, D_MODEL)], ssem[b]).wait()

    def process(i, b):
        wait_gathers(i, b)
        _scale_buf(rows[b])
        fire_scatter(i, b)

    gathers(0, 0)
    gathers(1, 1)
    process(0, 0)
    gathers(2, 2)
    for i in (1, 2):
        process(i, i % NBUF)
        drain_scatter((i + 2) % NBUF)
        gathers(i + 2, (i + 2) % NBUF)

    def outer(io, _):
        i0 = 3 + io * NBUF
        for t in range(NBUF):
            i = i0 + t
            b = t
            wait_gathers(i, b)
            _scale_buf(rows[b])
            fire_scatter(i, b)
            bn = (b + 2) % NBUF
            drain_scatter(bn)
            gathers(i + 2, bn)
        return ()

    lax.fori_loop(0, (rows_per_w - 5) // NBUF, outer, ())

    for i in (rows_per_w - 2, rows_per_w - 1):
        process(i, i % NBUF)
    for i in (rows_per_w - 3, rows_per_w - 2, rows_per_w - 1):
        drain_scatter(i % NBUF)


def kernel(x, table):
    n_seq, seq = x.shape
    assert seq == SEQ
    assert n_seq % NW == 0
    rows_per_w = n_seq // NW
    assert (rows_per_w - 5) % NBUF == 0

    t128 = jnp.pad(table, ((0, 0), (0, DPAD - D_MODEL)))

    mesh = plsc.VectorSubcoreMesh(
        core_axis_name="c", subcore_axis_name="s", num_cores=NC, num_subcores=NS
    )
    run = pl.kernel(
        functools.partial(_emb_body, rows_per_w),
        out_type=jax.ShapeDtypeStruct((n_seq, seq, DPAD), jnp.float32),
        mesh=mesh,
        scratch_types=(
            [pltpu.VMEM((rows_per_w, SEQ), jnp.int32)]
            + [pltpu.VMEM((SEQ, DPAD), jnp.float32) for _ in range(NBUF)]
            + [pltpu.SemaphoreType.DMA for _ in range(2 * NBUF)]
        ),
        compiler_params=pltpu.CompilerParams(use_tc_tiling_on_sc=False),
    )
    out128 = run(x, t128)
    return out128[:, :, :D_MODEL]
